# baseline (speedup 1.0000x reference)
import functools

import jax
import jax.numpy as jnp
from jax import lax
from jax.experimental import pallas as pl
from jax.experimental.pallas import tpu as pltpu
from jax.experimental.pallas import tpu_sc as plsc

DIM = 64
NC = 2
NS = 16
NW = NC * NS

CHUNK = 128


@functools.lru_cache(maxsize=None)
def _make_kernel(B: int):
    assert B % (NW * CHUNK) == 0
    b_per_w = B // NW
    n_chunks = b_per_w // CHUNK
    mesh = plsc.VectorSubcoreMesh(core_axis_name="c", subcore_axis_name="s")

    @functools.partial(
        pl.kernel,
        mesh=mesh,
        compiler_params=pltpu.CompilerParams(use_tc_tiling_on_sc=False),
        out_type=jax.ShapeDtypeStruct((B, DIM), jnp.float32),
        scratch_types=[
            pltpu.VMEM((CHUNK,), jnp.int32),
            pltpu.VMEM((CHUNK, DIM), jnp.float32),
            pltpu.SemaphoreType.DMA,
        ],
    )
    def emb(idx_hbm, table_hbm, out_hbm, idx_v, rows_v, sem):
        wid = lax.axis_index("s") * NC + lax.axis_index("c")
        base = wid * b_per_w

        def body(j, carry):
            off = base + j * CHUNK
            pltpu.sync_copy(idx_hbm.at[pl.ds(off, CHUNK)], idx_v)
            pltpu.async_copy(table_hbm.at[idx_v], rows_v, sem).wait()
            pltpu.sync_copy(rows_v, out_hbm.at[pl.ds(off, CHUNK)])
            return carry

        lax.fori_loop(0, n_chunks, body, 0)

    return emb


def kernel(X, table):
    rows, cols = X.shape
    idx = X.reshape(-1).astype(jnp.int32)
    out = _make_kernel(idx.shape[0])(idx, table)
    return out.reshape(rows, cols, DIM)


# speedup vs baseline: 1.1902x; 1.1902x over previous
import functools

import jax
import jax.numpy as jnp
from jax import lax
from jax.experimental import pallas as pl
from jax.experimental.pallas import tpu as pltpu
from jax.experimental.pallas import tpu_sc as plsc

DIM = 64
NC = 2
NS = 16
NW = NC * NS

CHUNK = 512


@functools.lru_cache(maxsize=None)
def _make_kernel(B: int):
    assert B % (NW * CHUNK) == 0
    b_per_w = B // NW
    n_chunks = b_per_w // CHUNK
    assert n_chunks % 2 == 0
    mesh = plsc.VectorSubcoreMesh(core_axis_name="c", subcore_axis_name="s")

    @functools.partial(
        pl.kernel,
        mesh=mesh,
        compiler_params=pltpu.CompilerParams(use_tc_tiling_on_sc=False),
        out_type=jax.ShapeDtypeStruct((B, DIM), jnp.float32),
        scratch_types=[
            pltpu.VMEM((n_chunks, CHUNK), jnp.int32),
            pltpu.VMEM((2, CHUNK, DIM), jnp.float32),
            pltpu.SemaphoreType.DMA,
            pltpu.SemaphoreType.DMA,
        ],
    )
    def emb(idx_hbm, table_hbm, out_hbm, idx_v, rows_v, gsem, wsem):
        wid = lax.axis_index("s") * NC + lax.axis_index("c")
        base = wid * b_per_w

        pltpu.sync_copy(idx_hbm.at[pl.ds(wid * n_chunks, n_chunks)], idx_v)

        def fire_gather(j, buf):
            pltpu.async_copy(table_hbm.at[idx_v.at[j]], rows_v.at[buf], gsem)

        def drain_gather(j, buf):
            pltpu.make_async_copy(
                table_hbm.at[idx_v.at[j]], rows_v.at[buf], gsem).wait()

        def fire_write(j, buf):
            pltpu.async_copy(
                rows_v.at[buf], out_hbm.at[pl.ds(base + j * CHUNK, CHUNK)], wsem)

        def drain_write(j, buf):
            pltpu.make_async_copy(
                rows_v.at[buf], out_hbm.at[pl.ds(base + j * CHUNK, CHUNK)],
                wsem).wait()

        fire_gather(0, 0)
        drain_gather(0, 0)
        fire_gather(1, 1)
        fire_write(0, 0)

        def body(p, carry):
            for b in range(2):
                j = 1 + 2 * p + b
                buf = (1 + b) % 2
                other = 1 - buf
                drain_gather(j, buf)
                drain_write(j - 1, other)
                fire_gather(j + 1, other)
                fire_write(j, buf)
            return carry

        lax.fori_loop(0, (n_chunks - 2) // 2, body, 0)

        j_last = n_chunks - 1
        drain_gather(j_last, 1)
        drain_write(j_last - 1, 0)
        fire_write(j_last, 1)
        drain_write(j_last, 1)

    return emb


def kernel(X, table):
    rows, cols = X.shape
    idx = X.reshape(-1, CHUNK).astype(jnp.int32)
    out = _make_kernel(idx.size)(idx, table)
    return out.reshape(rows, cols, DIM)
